# Initial kernel scaffold; baseline (speedup 1.0000x reference)
import jax
import jax.numpy as jnp
from jax.experimental import pallas as pl


def kernel(x, edge_index, W1, b1, W2, b2, W3, b3):
    raise NotImplementedError("write your pallas kernel here")


# speedup vs baseline: 15.7171x; 15.7171x over previous
import functools

import jax
import jax.numpy as jnp
from jax import lax
from jax.experimental import pallas as pl
from jax.experimental.pallas import tpu as pltpu
from jax.experimental.pallas import tpu_sc as plsc

N = 10000
E = 320000
D = 128

NC = 2
NS = 16
NW = NC * NS
EPW = E // NW
CHUNK = 80
NCHUNK = EPW // CHUNK
RPT = N // NS
N_PAD = 10240
RPT_PAD = N_PAD // NS

_sc_mesh = plsc.VectorSubcoreMesh(core_axis_name="c", subcore_axis_name="s")


@functools.partial(
    pl.kernel,
    out_type=jax.ShapeDtypeStruct((NC, N_PAD), jnp.float32),
    mesh=_sc_mesh,
    scratch_types=[
        pltpu.VMEM((NCHUNK, CHUNK), jnp.int32),
        pltpu.VMEM((CHUNK,), jnp.float32),
        pltpu.VMEM_SHARED((N_PAD,), jnp.float32),
    ],
)
def _deg_kernel(dst3_hbm, zeros_hbm, out_hbm, idx_v, ones_v, acc_sh):
    c = lax.axis_index("c")
    s = lax.axis_index("s")
    wid = c * NS + s
    pltpu.sync_copy(dst3_hbm.at[wid], idx_v)
    for j in range(CHUNK // 16):
        ones_v[pl.ds(j * 16, 16)] = jnp.ones((16,), jnp.float32)
    pltpu.sync_copy(zeros_hbm.at[pl.ds(s * RPT_PAD, RPT_PAD)],
                    acc_sh.at[pl.ds(s * RPT_PAD, RPT_PAD)])
    plsc.subcore_barrier()

    def body(i, carry):
        pltpu.sync_copy(ones_v, acc_sh.at[idx_v.at[i]], add=True)
        return carry

    lax.fori_loop(0, NCHUNK, body, 0)
    plsc.subcore_barrier()
    pltpu.sync_copy(acc_sh.at[pl.ds(s * RPT_PAD, RPT_PAD)],
                    out_hbm.at[c, pl.ds(s * RPT_PAD, RPT_PAD)])


@functools.partial(
    pl.kernel,
    out_type=jax.ShapeDtypeStruct((NC, N_PAD, D), jnp.float32),
    mesh=_sc_mesh,
    scratch_types=[
        pltpu.VMEM((NCHUNK, CHUNK), jnp.int32),
        pltpu.VMEM((NCHUNK, CHUNK), jnp.int32),
        pltpu.VMEM((CHUNK, D), jnp.float32),
        pltpu.VMEM_SHARED((N_PAD, D), jnp.float32),
        pltpu.SemaphoreType.DMA,
    ],
)
def _agg_kernel(g_hbm, src3_hbm, dst3_hbm, zeros_hbm, out_hbm,
                sidx_v, didx_v, rows_v, acc_sh, sem):
    c = lax.axis_index("c")
    s = lax.axis_index("s")
    wid = c * NS + s
    pltpu.sync_copy(src3_hbm.at[wid], sidx_v)
    pltpu.sync_copy(dst3_hbm.at[wid], didx_v)
    pltpu.sync_copy(zeros_hbm.at[pl.ds(s * RPT_PAD, RPT_PAD)],
                    acc_sh.at[pl.ds(s * RPT_PAD, RPT_PAD)])
    plsc.subcore_barrier()

    def body(i, carry):
        pltpu.async_copy(g_hbm.at[sidx_v.at[i]], rows_v, sem).wait()
        pltpu.sync_copy(rows_v, acc_sh.at[didx_v.at[i]], add=True)
        return carry

    lax.fori_loop(0, NCHUNK, body, 0)
    plsc.subcore_barrier()
    pltpu.sync_copy(acc_sh.at[pl.ds(s * RPT_PAD, RPT_PAD)],
                    out_hbm.at[c, pl.ds(s * RPT_PAD, RPT_PAD)])


_RB = 1000
_GRID = N // _RB


def _mm_scale_body(dis_ref, h_ref, w_ref, o_ref):
    o_ref[...] = dis_ref[...] * jnp.dot(h_ref[...], w_ref[...],
                                        preferred_element_type=jnp.float32)


_mm_scale = pl.pallas_call(
    _mm_scale_body,
    grid=(_GRID,),
    in_specs=[
        pl.BlockSpec((_RB, 1), lambda i: (i, 0)),
        pl.BlockSpec((_RB, D), lambda i: (i, 0)),
        pl.BlockSpec((D, D), lambda i: (0, 0)),
    ],
    out_specs=pl.BlockSpec((_RB, D), lambda i: (i, 0)),
    out_shape=jax.ShapeDtypeStruct((N, D), jnp.float32),
)


def _post_mm_body(p_ref, g_ref, dis_ref, b_ref, w_ref, o_ref):
    h = dis_ref[...] * (p_ref[0] + p_ref[1] + g_ref[...]) + b_ref[...]
    h = jnp.maximum(h, 0.0)
    o_ref[...] = dis_ref[...] * jnp.dot(h, w_ref[...],
                                        preferred_element_type=jnp.float32)


_post_mm = pl.pallas_call(
    _post_mm_body,
    grid=(_GRID,),
    in_specs=[
        pl.BlockSpec((NC, _RB, D), lambda i: (0, i, 0)),
        pl.BlockSpec((_RB, D), lambda i: (i, 0)),
        pl.BlockSpec((_RB, 1), lambda i: (i, 0)),
        pl.BlockSpec((1, D), lambda i: (0, 0)),
        pl.BlockSpec((D, D), lambda i: (0, 0)),
    ],
    out_specs=pl.BlockSpec((_RB, D), lambda i: (i, 0)),
    out_shape=jax.ShapeDtypeStruct((N, D), jnp.float32),
)


def _final_body(p_ref, g_ref, dis_ref, b_ref, x_ref, o_ref):
    o_ref[...] = (dis_ref[...] * (p_ref[0] + p_ref[1] + g_ref[...])
                  + b_ref[...] + x_ref[...])


_final = pl.pallas_call(
    _final_body,
    grid=(_GRID,),
    in_specs=[
        pl.BlockSpec((NC, _RB, D), lambda i: (0, i, 0)),
        pl.BlockSpec((_RB, D), lambda i: (i, 0)),
        pl.BlockSpec((_RB, 1), lambda i: (i, 0)),
        pl.BlockSpec((1, D), lambda i: (0, 0)),
        pl.BlockSpec((_RB, D), lambda i: (i, 0)),
    ],
    out_specs=pl.BlockSpec((_RB, D), lambda i: (i, 0)),
    out_shape=jax.ShapeDtypeStruct((N, D), jnp.float32),
)


def kernel(x, edge_index, W1, b1, W2, b2, W3, b3):
    src3 = edge_index[0].reshape(NW, NCHUNK, CHUNK)
    dst3 = edge_index[1].reshape(NW, NCHUNK, CHUNK)
    zeros1 = jnp.zeros((N_PAD,), jnp.float32)
    zeros2 = jnp.zeros((N_PAD, D), jnp.float32)

    degp = _deg_kernel(dst3, zeros1)
    dis = lax.rsqrt(degp[0, :N] + degp[1, :N] + 1.0).reshape(N, 1)

    g = _mm_scale(dis, x, W1)
    p = _agg_kernel(g, src3, dst3, zeros2)
    g = _post_mm(p, g, dis, b1.reshape(1, D), W2)
    p = _agg_kernel(g, src3, dst3, zeros2)
    g = _post_mm(p, g, dis, b2.reshape(1, D), W3)
    p = _agg_kernel(g, src3, dst3, zeros2)
    return _final(p, g, dis, b3.reshape(1, D), x)


# speedup vs baseline: 21.4155x; 1.3626x over previous
import functools

import jax
import jax.numpy as jnp
from jax import lax
from jax.experimental import pallas as pl
from jax.experimental.pallas import tpu as pltpu
from jax.experimental.pallas import tpu_sc as plsc

N = 10000
E = 320000
D = 128

NC = 2
NS = 16
NW = NC * NS
EPW = E // NW
CHUNK = 80
NCHUNK = EPW // CHUNK
RPT = N // NS
N_PAD = 10240
RPT_PAD = N_PAD // NS

_sc_mesh = plsc.VectorSubcoreMesh(core_axis_name="c", subcore_axis_name="s")


@functools.partial(
    pl.kernel,
    out_type=jax.ShapeDtypeStruct((NC, N_PAD), jnp.float32),
    mesh=_sc_mesh,
    scratch_types=[
        pltpu.VMEM((NCHUNK, CHUNK), jnp.int32),
        pltpu.VMEM((CHUNK,), jnp.float32),
        pltpu.VMEM_SHARED((N_PAD,), jnp.float32),
    ],
)
def _deg_kernel(dst3_hbm, zeros_hbm, out_hbm, idx_v, ones_v, acc_sh):
    c = lax.axis_index("c")
    s = lax.axis_index("s")
    wid = c * NS + s
    pltpu.sync_copy(dst3_hbm.at[wid], idx_v)
    for j in range(CHUNK // 16):
        ones_v[pl.ds(j * 16, 16)] = jnp.ones((16,), jnp.float32)
    pltpu.sync_copy(zeros_hbm.at[pl.ds(s * RPT_PAD, RPT_PAD)],
                    acc_sh.at[pl.ds(s * RPT_PAD, RPT_PAD)])
    plsc.subcore_barrier()

    def body(i, carry):
        pltpu.sync_copy(ones_v, acc_sh.at[idx_v.at[i]], add=True)
        return carry

    lax.fori_loop(0, NCHUNK, body, 0)
    plsc.subcore_barrier()
    pltpu.sync_copy(acc_sh.at[pl.ds(s * RPT_PAD, RPT_PAD)],
                    out_hbm.at[c, pl.ds(s * RPT_PAD, RPT_PAD)])


ACH = 125
WCH = 4
NWIN = EPW // (ACH * WCH)


@functools.partial(
    pl.kernel,
    out_type=jax.ShapeDtypeStruct((NC, N_PAD, D), jnp.float32),
    mesh=_sc_mesh,
    scratch_types=[
        pltpu.VMEM((2, WCH, ACH), jnp.int32),
        pltpu.VMEM((2, WCH, ACH), jnp.int32),
        pltpu.VMEM((2, ACH, D), jnp.float32),
        pltpu.VMEM_SHARED((N_PAD, D), jnp.float32),
        pltpu.SemaphoreType.DMA((2,)),
        pltpu.SemaphoreType.DMA((2,)),
        pltpu.SemaphoreType.DMA((2,)),
    ],
)
def _agg_kernel(g_hbm, src4_hbm, dst4_hbm, zeros_hbm, out_hbm,
                sidx_v, didx_v, rows_v, acc_sh, gsem, ssem, wsem):
    c = lax.axis_index("c")
    s = lax.axis_index("s")
    wid = c * NS + s

    def load_win(w, wb):
        pltpu.async_copy(src4_hbm.at[wid, w], sidx_v.at[wb], wsem.at[0])
        pltpu.async_copy(dst4_hbm.at[wid, w], didx_v.at[wb], wsem.at[1])

    def wait_win(w, wb):
        pltpu.make_async_copy(src4_hbm.at[wid, w], sidx_v.at[wb],
                              wsem.at[0]).wait()
        pltpu.make_async_copy(dst4_hbm.at[wid, w], didx_v.at[wb],
                              wsem.at[1]).wait()

    def start_gather(wb, k, b):
        pltpu.async_copy(g_hbm.at[sidx_v.at[wb, k]], rows_v.at[b],
                         gsem.at[b])

    def wait_gather(wb, k, b):
        pltpu.make_async_copy(g_hbm.at[sidx_v.at[wb, k]], rows_v.at[b],
                              gsem.at[b]).wait()

    def start_scatter(wb, k, b):
        pltpu.async_copy(rows_v.at[b], acc_sh.at[didx_v.at[wb, k]],
                         ssem.at[b], add=True)

    def wait_scatter(wb, k, b):
        pltpu.make_async_copy(rows_v.at[b], acc_sh.at[didx_v.at[wb, k]],
                              ssem.at[b]).wait()

    load_win(0, 0)
    pltpu.sync_copy(zeros_hbm.at[pl.ds(s * RPT_PAD, RPT_PAD)],
                    acc_sh.at[pl.ds(s * RPT_PAD, RPT_PAD)])
    plsc.subcore_barrier()
    wait_win(0, 0)
    start_gather(0, 0, 0)
    start_gather(0, 1, 1)

    def outer(w, carry):
        wb = w % 2
        wbn = (w + 1) % 2
        nw = w + 1

        @pl.when(nw < NWIN)
        def _():
            load_win(nw, wbn)

        wait_gather(wb, 0, 0)
        start_scatter(wb, 0, 0)
        wait_gather(wb, 1, 1)
        start_scatter(wb, 1, 1)
        wait_scatter(wb, 0, 0)
        start_gather(wb, 2, 0)
        wait_scatter(wb, 1, 1)
        start_gather(wb, 3, 1)
        wait_gather(wb, 2, 0)
        start_scatter(wb, 2, 0)
        wait_gather(wb, 3, 1)
        start_scatter(wb, 3, 1)
        wait_scatter(wb, 2, 0)

        @pl.when(nw < NWIN)
        def _():
            wait_win(nw, wbn)
            start_gather(wbn, 0, 0)

        wait_scatter(wb, 3, 1)

        @pl.when(nw < NWIN)
        def _():
            start_gather(wbn, 1, 1)

        return carry

    lax.fori_loop(0, NWIN, outer, 0)
    plsc.subcore_barrier()
    pltpu.sync_copy(acc_sh.at[pl.ds(s * RPT_PAD, RPT_PAD)],
                    out_hbm.at[c, pl.ds(s * RPT_PAD, RPT_PAD)])


_RB = 1000
_GRID = N // _RB


def _mm_scale_body(dis_ref, h_ref, w_ref, o_ref):
    o_ref[...] = dis_ref[...] * jnp.dot(h_ref[...], w_ref[...],
                                        preferred_element_type=jnp.float32)


_mm_scale = pl.pallas_call(
    _mm_scale_body,
    grid=(_GRID,),
    in_specs=[
        pl.BlockSpec((_RB, 1), lambda i: (i, 0)),
        pl.BlockSpec((_RB, D), lambda i: (i, 0)),
        pl.BlockSpec((D, D), lambda i: (0, 0)),
    ],
    out_specs=pl.BlockSpec((_RB, D), lambda i: (i, 0)),
    out_shape=jax.ShapeDtypeStruct((N, D), jnp.float32),
)


def _post_mm_body(p_ref, g_ref, dis_ref, b_ref, w_ref, o_ref):
    h = dis_ref[...] * (p_ref[0] + p_ref[1] + g_ref[...]) + b_ref[...]
    h = jnp.maximum(h, 0.0)
    o_ref[...] = dis_ref[...] * jnp.dot(h, w_ref[...],
                                        preferred_element_type=jnp.float32)


_post_mm = pl.pallas_call(
    _post_mm_body,
    grid=(_GRID,),
    in_specs=[
        pl.BlockSpec((NC, _RB, D), lambda i: (0, i, 0)),
        pl.BlockSpec((_RB, D), lambda i: (i, 0)),
        pl.BlockSpec((_RB, 1), lambda i: (i, 0)),
        pl.BlockSpec((1, D), lambda i: (0, 0)),
        pl.BlockSpec((D, D), lambda i: (0, 0)),
    ],
    out_specs=pl.BlockSpec((_RB, D), lambda i: (i, 0)),
    out_shape=jax.ShapeDtypeStruct((N, D), jnp.float32),
)


def _final_body(p_ref, g_ref, dis_ref, b_ref, x_ref, o_ref):
    o_ref[...] = (dis_ref[...] * (p_ref[0] + p_ref[1] + g_ref[...])
                  + b_ref[...] + x_ref[...])


_final = pl.pallas_call(
    _final_body,
    grid=(_GRID,),
    in_specs=[
        pl.BlockSpec((NC, _RB, D), lambda i: (0, i, 0)),
        pl.BlockSpec((_RB, D), lambda i: (i, 0)),
        pl.BlockSpec((_RB, 1), lambda i: (i, 0)),
        pl.BlockSpec((1, D), lambda i: (0, 0)),
        pl.BlockSpec((_RB, D), lambda i: (i, 0)),
    ],
    out_specs=pl.BlockSpec((_RB, D), lambda i: (i, 0)),
    out_shape=jax.ShapeDtypeStruct((N, D), jnp.float32),
)


def kernel(x, edge_index, W1, b1, W2, b2, W3, b3):
    dst3 = edge_index[1].reshape(NW, NCHUNK, CHUNK)
    src4 = edge_index[0].reshape(NW, NWIN, WCH, ACH)
    dst4 = edge_index[1].reshape(NW, NWIN, WCH, ACH)
    zeros1 = jnp.zeros((N_PAD,), jnp.float32)
    zeros2 = jnp.zeros((N_PAD, D), jnp.float32)

    degp = _deg_kernel(dst3, zeros1)
    dis = lax.rsqrt(degp[0, :N] + degp[1, :N] + 1.0).reshape(N, 1)

    g = _mm_scale(dis, x, W1)
    p = _agg_kernel(g, src4, dst4, zeros2)
    g = _post_mm(p, g, dis, b1.reshape(1, D), W2)
    p = _agg_kernel(g, src4, dst4, zeros2)
    g = _post_mm(p, g, dis, b2.reshape(1, D), W3)
    p = _agg_kernel(g, src4, dst4, zeros2)
    return _final(p, g, dis, b3.reshape(1, D), x)


# speedup vs baseline: 28.9013x; 1.3495x over previous
import functools

import jax
import jax.numpy as jnp
from jax import lax
from jax.experimental import pallas as pl
from jax.experimental.pallas import tpu as pltpu
from jax.experimental.pallas import tpu_sc as plsc

N = 10000
E = 320000
D = 128

NC = 2
NS = 16
NW = NC * NS
EPW = E // NW
CHUNK = 80
NCHUNK = EPW // CHUNK
RPT = N // NS
N_PAD = 10240
RPT_PAD = N_PAD // NS

_sc_mesh = plsc.VectorSubcoreMesh(core_axis_name="c", subcore_axis_name="s")


@functools.partial(
    pl.kernel,
    out_type=jax.ShapeDtypeStruct((NC, N_PAD), jnp.float32),
    mesh=_sc_mesh,
    scratch_types=[
        pltpu.VMEM((NCHUNK, CHUNK), jnp.int32),
        pltpu.VMEM((CHUNK,), jnp.float32),
        pltpu.VMEM_SHARED((N_PAD,), jnp.float32),
    ],
)
def _deg_kernel(dst3_hbm, zeros_hbm, out_hbm, idx_v, ones_v, acc_sh):
    c = lax.axis_index("c")
    s = lax.axis_index("s")
    wid = c * NS + s
    pltpu.sync_copy(dst3_hbm.at[wid], idx_v)
    for j in range(CHUNK // 16):
        ones_v[pl.ds(j * 16, 16)] = jnp.ones((16,), jnp.float32)
    pltpu.sync_copy(zeros_hbm.at[pl.ds(s * RPT_PAD, RPT_PAD)],
                    acc_sh.at[pl.ds(s * RPT_PAD, RPT_PAD)])
    plsc.subcore_barrier()

    def body(i, carry):
        pltpu.sync_copy(ones_v, acc_sh.at[idx_v.at[i]], add=True)
        return carry

    lax.fori_loop(0, NCHUNK, body, 0)
    plsc.subcore_barrier()
    pltpu.sync_copy(acc_sh.at[pl.ds(s * RPT_PAD, RPT_PAD)],
                    out_hbm.at[c, pl.ds(s * RPT_PAD, RPT_PAD)])


ACH = 50
ANC = EPW // ACH
WCH = 10
NWIN = ANC // WCH
NBUF = 5


@functools.partial(
    pl.kernel,
    out_type=jax.ShapeDtypeStruct((NC, N_PAD, D), jnp.float32),
    mesh=_sc_mesh,
    scratch_types=[
        pltpu.VMEM((2, WCH, ACH), jnp.int32),
        pltpu.VMEM((2, WCH, ACH), jnp.int32),
        pltpu.VMEM((NBUF, ACH, D), jnp.float32),
        pltpu.VMEM_SHARED((N_PAD, D), jnp.float32),
        pltpu.SemaphoreType.DMA((NBUF,)),
        pltpu.SemaphoreType.DMA((NBUF,)),
        pltpu.SemaphoreType.DMA((2,)),
    ],
)
def _agg_kernel(g_hbm, src4_hbm, dst4_hbm, zeros_hbm, out_hbm,
                sidx_v, didx_v, rows_v, acc_sh, gsem, ssem, wsem):
    c = lax.axis_index("c")
    s = lax.axis_index("s")
    wid = c * NS + s

    def load_win(w, wb):
        pltpu.async_copy(src4_hbm.at[wid, w], sidx_v.at[wb], wsem.at[0])
        pltpu.async_copy(dst4_hbm.at[wid, w], didx_v.at[wb], wsem.at[1])

    def wait_win(w, wb):
        pltpu.make_async_copy(src4_hbm.at[wid, w], sidx_v.at[wb],
                              wsem.at[0]).wait()
        pltpu.make_async_copy(dst4_hbm.at[wid, w], didx_v.at[wb],
                              wsem.at[1]).wait()

    def _wk(i):
        return lax.rem(lax.div(i, WCH), 2), lax.rem(i, WCH)

    def start_gather(i):
        wb, k = _wk(i)
        b = lax.rem(i, NBUF)
        pltpu.async_copy(g_hbm.at[sidx_v.at[wb, k]], rows_v.at[b],
                         gsem.at[b])

    def wait_gather(i):
        wb, k = _wk(i)
        b = lax.rem(i, NBUF)
        pltpu.make_async_copy(g_hbm.at[sidx_v.at[wb, k]], rows_v.at[b],
                              gsem.at[b]).wait()

    def start_scatter(i):
        wb, k = _wk(i)
        b = lax.rem(i, NBUF)
        pltpu.async_copy(rows_v.at[b], acc_sh.at[didx_v.at[wb, k]],
                         ssem.at[b], add=True)

    def wait_scatter(i):
        wb, k = _wk(i)
        b = lax.rem(i, NBUF)
        pltpu.make_async_copy(rows_v.at[b], acc_sh.at[didx_v.at[wb, k]],
                              ssem.at[b]).wait()

    load_win(0, 0)
    pltpu.sync_copy(zeros_hbm.at[pl.ds(s * RPT_PAD, RPT_PAD)],
                    acc_sh.at[pl.ds(s * RPT_PAD, RPT_PAD)])
    plsc.subcore_barrier()
    wait_win(0, 0)
    load_win(1, 1)
    for jj in range(NBUF - 1):
        start_gather(jnp.int32(jj))

    def body(i, carry):
        wait_gather(i)
        start_scatter(i)

        @pl.when(i >= 1)
        def _():
            wait_scatter(i - 1)

        @pl.when(jnp.logical_and(i >= 1, lax.rem(i, WCH) == 0))
        def _():
            w = lax.div(i, WCH)

            @pl.when(w + 1 < NWIN)
            def _():
                load_win(w + 1, lax.rem(w + 1, 2))

        j = i + NBUF - 1

        @pl.when(j < ANC)
        def _():
            @pl.when(lax.rem(j, WCH) == 0)
            def _():
                wj = lax.div(j, WCH)
                wait_win(wj, lax.rem(wj, 2))

            start_gather(j)

        return carry

    lax.fori_loop(0, ANC, body, jnp.int32(0))
    wait_scatter(jnp.int32(ANC - 1))
    plsc.subcore_barrier()
    pltpu.sync_copy(acc_sh.at[pl.ds(s * RPT_PAD, RPT_PAD)],
                    out_hbm.at[c, pl.ds(s * RPT_PAD, RPT_PAD)])


_RB = 1000
_GRID = N // _RB


def _mm_scale_body(dis_ref, h_ref, w_ref, o_ref):
    o_ref[...] = dis_ref[...] * jnp.dot(h_ref[...], w_ref[...],
                                        preferred_element_type=jnp.float32)


_mm_scale = pl.pallas_call(
    _mm_scale_body,
    grid=(_GRID,),
    in_specs=[
        pl.BlockSpec((_RB, 1), lambda i: (i, 0)),
        pl.BlockSpec((_RB, D), lambda i: (i, 0)),
        pl.BlockSpec((D, D), lambda i: (0, 0)),
    ],
    out_specs=pl.BlockSpec((_RB, D), lambda i: (i, 0)),
    out_shape=jax.ShapeDtypeStruct((N, D), jnp.float32),
)


def _post_mm_body(p_ref, g_ref, dis_ref, b_ref, w_ref, o_ref):
    h = dis_ref[...] * (p_ref[0] + p_ref[1] + g_ref[...]) + b_ref[...]
    h = jnp.maximum(h, 0.0)
    o_ref[...] = dis_ref[...] * jnp.dot(h, w_ref[...],
                                        preferred_element_type=jnp.float32)


_post_mm = pl.pallas_call(
    _post_mm_body,
    grid=(_GRID,),
    in_specs=[
        pl.BlockSpec((NC, _RB, D), lambda i: (0, i, 0)),
        pl.BlockSpec((_RB, D), lambda i: (i, 0)),
        pl.BlockSpec((_RB, 1), lambda i: (i, 0)),
        pl.BlockSpec((1, D), lambda i: (0, 0)),
        pl.BlockSpec((D, D), lambda i: (0, 0)),
    ],
    out_specs=pl.BlockSpec((_RB, D), lambda i: (i, 0)),
    out_shape=jax.ShapeDtypeStruct((N, D), jnp.float32),
)


def _final_body(p_ref, g_ref, dis_ref, b_ref, x_ref, o_ref):
    o_ref[...] = (dis_ref[...] * (p_ref[0] + p_ref[1] + g_ref[...])
                  + b_ref[...] + x_ref[...])


_final = pl.pallas_call(
    _final_body,
    grid=(_GRID,),
    in_specs=[
        pl.BlockSpec((NC, _RB, D), lambda i: (0, i, 0)),
        pl.BlockSpec((_RB, D), lambda i: (i, 0)),
        pl.BlockSpec((_RB, 1), lambda i: (i, 0)),
        pl.BlockSpec((1, D), lambda i: (0, 0)),
        pl.BlockSpec((_RB, D), lambda i: (i, 0)),
    ],
    out_specs=pl.BlockSpec((_RB, D), lambda i: (i, 0)),
    out_shape=jax.ShapeDtypeStruct((N, D), jnp.float32),
)


def kernel(x, edge_index, W1, b1, W2, b2, W3, b3):
    dst3 = edge_index[1].reshape(NW, NCHUNK, CHUNK)
    src4 = edge_index[0].reshape(NW, NWIN, WCH, ACH)
    dst4 = edge_index[1].reshape(NW, NWIN, WCH, ACH)
    zeros1 = jnp.zeros((N_PAD,), jnp.float32)
    zeros2 = jnp.zeros((N_PAD, D), jnp.float32)

    degp = _deg_kernel(dst3, zeros1)
    dis = lax.rsqrt(degp[0, :N] + degp[1, :N] + 1.0).reshape(N, 1)

    g = _mm_scale(dis, x, W1)
    p = _agg_kernel(g, src4, dst4, zeros2)
    g = _post_mm(p, g, dis, b1.reshape(1, D), W2)
    p = _agg_kernel(g, src4, dst4, zeros2)
    g = _post_mm(p, g, dis, b2.reshape(1, D), W3)
    p = _agg_kernel(g, src4, dst4, zeros2)
    return _final(p, g, dis, b3.reshape(1, D), x)


# speedup vs baseline: 29.5160x; 1.0213x over previous
import functools

import jax
import jax.numpy as jnp
from jax import lax
from jax.experimental import pallas as pl
from jax.experimental.pallas import tpu as pltpu
from jax.experimental.pallas import tpu_sc as plsc

N = 10000
E = 320000
D = 128

NC = 2
NS = 16
NW = NC * NS
EPW = E // NW
CHUNK = 80
NCHUNK = EPW // CHUNK
RPT = N // NS
N_PAD = 10240
RPT_PAD = N_PAD // NS

_sc_mesh = plsc.VectorSubcoreMesh(core_axis_name="c", subcore_axis_name="s")


@functools.partial(
    pl.kernel,
    out_type=jax.ShapeDtypeStruct((NC, N_PAD), jnp.float32),
    mesh=_sc_mesh,
    scratch_types=[
        pltpu.VMEM((NCHUNK, CHUNK), jnp.int32),
        pltpu.VMEM((CHUNK,), jnp.float32),
        pltpu.VMEM_SHARED((N_PAD,), jnp.float32),
        pltpu.SemaphoreType.DMA((8,)),
    ],
)
def _deg_kernel(dst3_hbm, zeros_hbm, out_hbm, idx_v, ones_v, acc_sh, ssem):
    c = lax.axis_index("c")
    s = lax.axis_index("s")
    wid = c * NS + s
    pltpu.sync_copy(dst3_hbm.at[wid], idx_v)
    for j in range(CHUNK // 16):
        ones_v[pl.ds(j * 16, 16)] = jnp.ones((16,), jnp.float32)
    pltpu.sync_copy(zeros_hbm.at[pl.ds(s * RPT_PAD, RPT_PAD)],
                    acc_sh.at[pl.ds(s * RPT_PAD, RPT_PAD)])
    plsc.subcore_barrier()

    def start_sc(i):
        pltpu.async_copy(ones_v, acc_sh.at[idx_v.at[i]],
                         ssem.at[lax.rem(i, 8)], add=True)

    def wait_sc(i):
        pltpu.make_async_copy(ones_v, acc_sh.at[idx_v.at[i]],
                              ssem.at[lax.rem(i, 8)]).wait()

    def body(i, carry):
        @pl.when(i >= 8)
        def _():
            wait_sc(i - 8)

        start_sc(i)
        return carry

    lax.fori_loop(0, NCHUNK, body, jnp.int32(0))

    def drain(i, carry):
        wait_sc(i)
        return carry

    lax.fori_loop(NCHUNK - 8, NCHUNK, drain, jnp.int32(0))
    plsc.subcore_barrier()
    pltpu.sync_copy(acc_sh.at[pl.ds(s * RPT_PAD, RPT_PAD)],
                    out_hbm.at[c, pl.ds(s * RPT_PAD, RPT_PAD)])


ACH = 50
ANC = EPW // ACH
WCH = 10
NWIN = ANC // WCH
NBUF = 5


@functools.partial(
    pl.kernel,
    out_type=jax.ShapeDtypeStruct((NC, N_PAD, D), jnp.float32),
    mesh=_sc_mesh,
    scratch_types=[
        pltpu.VMEM((2, WCH, ACH), jnp.int32),
        pltpu.VMEM((2, WCH, ACH), jnp.int32),
        pltpu.VMEM((NBUF, ACH, D), jnp.float32),
        pltpu.VMEM_SHARED((N_PAD, D), jnp.float32),
        pltpu.SemaphoreType.DMA((NBUF,)),
        pltpu.SemaphoreType.DMA((NBUF,)),
        pltpu.SemaphoreType.DMA((2,)),
    ],
)
def _agg_kernel(g_hbm, src4_hbm, dst4_hbm, zeros_hbm, out_hbm,
                sidx_v, didx_v, rows_v, acc_sh, gsem, ssem, wsem):
    c = lax.axis_index("c")
    s = lax.axis_index("s")
    wid = c * NS + s

    def load_win(w, wb):
        pltpu.async_copy(src4_hbm.at[wid, w], sidx_v.at[wb], wsem.at[0])
        pltpu.async_copy(dst4_hbm.at[wid, w], didx_v.at[wb], wsem.at[1])

    def wait_win(w, wb):
        pltpu.make_async_copy(src4_hbm.at[wid, w], sidx_v.at[wb],
                              wsem.at[0]).wait()
        pltpu.make_async_copy(dst4_hbm.at[wid, w], didx_v.at[wb],
                              wsem.at[1]).wait()

    def _wk(i):
        return lax.rem(lax.div(i, WCH), 2), lax.rem(i, WCH)

    def start_gather(i):
        wb, k = _wk(i)
        b = lax.rem(i, NBUF)
        pltpu.async_copy(g_hbm.at[sidx_v.at[wb, k]], rows_v.at[b],
                         gsem.at[b])

    def wait_gather(i):
        wb, k = _wk(i)
        b = lax.rem(i, NBUF)
        pltpu.make_async_copy(g_hbm.at[sidx_v.at[wb, k]], rows_v.at[b],
                              gsem.at[b]).wait()

    def start_scatter(i):
        wb, k = _wk(i)
        b = lax.rem(i, NBUF)
        pltpu.async_copy(rows_v.at[b], acc_sh.at[didx_v.at[wb, k]],
                         ssem.at[b], add=True)

    def wait_scatter(i):
        wb, k = _wk(i)
        b = lax.rem(i, NBUF)
        pltpu.make_async_copy(rows_v.at[b], acc_sh.at[didx_v.at[wb, k]],
                              ssem.at[b]).wait()

    load_win(0, 0)
    pltpu.sync_copy(zeros_hbm.at[pl.ds(s * RPT_PAD, RPT_PAD)],
                    acc_sh.at[pl.ds(s * RPT_PAD, RPT_PAD)])
    plsc.subcore_barrier()
    wait_win(0, 0)
    load_win(1, 1)
    for jj in range(NBUF - 1):
        start_gather(jnp.int32(jj))

    def body(i, carry):
        wait_gather(i)
        start_scatter(i)

        @pl.when(i >= 1)
        def _():
            wait_scatter(i - 1)

        @pl.when(jnp.logical_and(i >= 1, lax.rem(i, WCH) == 0))
        def _():
            w = lax.div(i, WCH)

            @pl.when(w + 1 < NWIN)
            def _():
                load_win(w + 1, lax.rem(w + 1, 2))

        j = i + NBUF - 1

        @pl.when(j < ANC)
        def _():
            @pl.when(lax.rem(j, WCH) == 0)
            def _():
                wj = lax.div(j, WCH)
                wait_win(wj, lax.rem(wj, 2))

            start_gather(j)

        return carry

    lax.fori_loop(0, ANC, body, jnp.int32(0))
    wait_scatter(jnp.int32(ANC - 1))
    plsc.subcore_barrier()
    pltpu.sync_copy(acc_sh.at[pl.ds(s * RPT_PAD, RPT_PAD)],
                    out_hbm.at[c, pl.ds(s * RPT_PAD, RPT_PAD)])


_RB = 1000
_GRID = N // _RB


def _mm_scale_body(dis_ref, h_ref, w_ref, o_ref):
    o_ref[...] = dis_ref[...] * jnp.dot(h_ref[...], w_ref[...],
                                        preferred_element_type=jnp.float32)


_mm_scale = pl.pallas_call(
    _mm_scale_body,
    grid=(_GRID,),
    in_specs=[
        pl.BlockSpec((_RB, 1), lambda i: (i, 0)),
        pl.BlockSpec((_RB, D), lambda i: (i, 0)),
        pl.BlockSpec((D, D), lambda i: (0, 0)),
    ],
    out_specs=pl.BlockSpec((_RB, D), lambda i: (i, 0)),
    out_shape=jax.ShapeDtypeStruct((N, D), jnp.float32),
)


def _post_mm_body(p_ref, g_ref, dis_ref, b_ref, w_ref, o_ref):
    h = dis_ref[...] * (p_ref[0] + p_ref[1] + g_ref[...]) + b_ref[...]
    h = jnp.maximum(h, 0.0)
    o_ref[...] = dis_ref[...] * jnp.dot(h, w_ref[...],
                                        preferred_element_type=jnp.float32)


_post_mm = pl.pallas_call(
    _post_mm_body,
    grid=(_GRID,),
    in_specs=[
        pl.BlockSpec((NC, _RB, D), lambda i: (0, i, 0)),
        pl.BlockSpec((_RB, D), lambda i: (i, 0)),
        pl.BlockSpec((_RB, 1), lambda i: (i, 0)),
        pl.BlockSpec((1, D), lambda i: (0, 0)),
        pl.BlockSpec((D, D), lambda i: (0, 0)),
    ],
    out_specs=pl.BlockSpec((_RB, D), lambda i: (i, 0)),
    out_shape=jax.ShapeDtypeStruct((N, D), jnp.float32),
)


def _final_body(p_ref, g_ref, dis_ref, b_ref, x_ref, o_ref):
    o_ref[...] = (dis_ref[...] * (p_ref[0] + p_ref[1] + g_ref[...])
                  + b_ref[...] + x_ref[...])


_final = pl.pallas_call(
    _final_body,
    grid=(_GRID,),
    in_specs=[
        pl.BlockSpec((NC, _RB, D), lambda i: (0, i, 0)),
        pl.BlockSpec((_RB, D), lambda i: (i, 0)),
        pl.BlockSpec((_RB, 1), lambda i: (i, 0)),
        pl.BlockSpec((1, D), lambda i: (0, 0)),
        pl.BlockSpec((_RB, D), lambda i: (i, 0)),
    ],
    out_specs=pl.BlockSpec((_RB, D), lambda i: (i, 0)),
    out_shape=jax.ShapeDtypeStruct((N, D), jnp.float32),
)


def kernel(x, edge_index, W1, b1, W2, b2, W3, b3):
    dst3 = edge_index[1].reshape(NW, NCHUNK, CHUNK)
    src4 = edge_index[0].reshape(NW, NWIN, WCH, ACH)
    dst4 = edge_index[1].reshape(NW, NWIN, WCH, ACH)
    zeros1 = jnp.zeros((N_PAD,), jnp.float32)
    zeros2 = jnp.zeros((N_PAD, D), jnp.float32)

    degp = _deg_kernel(dst3, zeros1)
    dis = lax.rsqrt(degp[0, :N] + degp[1, :N] + 1.0).reshape(N, 1)

    g = _mm_scale(dis, x, W1)
    p = _agg_kernel(g, src4, dst4, zeros2)
    g = _post_mm(p, g, dis, b1.reshape(1, D), W2)
    p = _agg_kernel(g, src4, dst4, zeros2)
    g = _post_mm(p, g, dis, b2.reshape(1, D), W3)
    p = _agg_kernel(g, src4, dst4, zeros2)
    return _final(p, g, dis, b3.reshape(1, D), x)
